# baseline (speedup 1.0000x reference)
import jax
import jax.numpy as jnp
from jax import lax
from jax.experimental import pallas as pl
from jax.experimental.pallas import tpu as pltpu
from jax.experimental.pallas import tpu_sc as plsc

N_USERS = 1000000
N_ITEMS = 1000000
EMBED_DIM = 16
BATCH = 16384
LAM = 1.0

NC = 2
NS = 16
L = 16
NW = NC * NS
BPW = BATCH // NW
NCHUNK = 4
CHUNK = BPW // NCHUNK
NGROUP = BPW // L

_LN2 = 0.69314718
_SQRT2 = 1.4142135


def _ln(x):
    xi = lax.bitcast_convert_type(x, jnp.int32)
    e = lax.shift_right_arithmetic(xi, 23) - 127
    mi = jnp.bitwise_or(jnp.bitwise_and(xi, 0x007FFFFF), 0x3F800000)
    m = lax.bitcast_convert_type(mi, jnp.float32)
    big = m > _SQRT2
    m = jnp.where(big, m * 0.5, m)
    e = jnp.where(big, e + 1, e)
    ef = e.astype(jnp.float32)
    s = (m - 1.0) / (m + 1.0)
    z = s * s
    p = 2.0 * s * (1.0 + z * (1.0 / 3.0 + z * (0.2 + z * (1.0 / 7.0 + z * (1.0 / 9.0)))))
    return ef * _LN2 + p


def _sigmoid(g):
    return 1.0 / (1.0 + jnp.exp(-g))


def _sc_body(users_hbm, pos_hbm, neg_hbm, utab_hbm, itab_hbm,
             mu_hbm, mi_hbm,
             pos_out, neg_out, reg_out,
             u_idx, p_idx, n_idx,
             us_idx, ps_idx, ns_idx,
             u_rows, p_rows, n_rows,
             mu_rows, mp_rows, mn_rows,
             pos_v, neg_v, reg_v, sem):
    wid = lax.axis_index("s") * NC + lax.axis_index("c")
    base = wid * BPW

    for j in range(NCHUNK):
        off = base + j * CHUNK
        pltpu.sync_copy(users_hbm.at[pl.ds(off, CHUNK)], u_idx.at[j])
        pltpu.sync_copy(pos_hbm.at[pl.ds(off, CHUNK)], p_idx.at[j])
        pltpu.sync_copy(neg_hbm.at[pl.ds(off, CHUNK)], n_idx.at[j])

    for j in range(NCHUNK):
        for r in range(CHUNK // L):
            sl = pl.ds(r * L, L)
            us_idx[j, sl] = lax.shift_right_logical(u_idx[j, sl], 4)
            ps_idx[j, sl] = lax.shift_right_logical(p_idx[j, sl], 4)
            ns_idx[j, sl] = lax.shift_right_logical(n_idx[j, sl], 4)

    copies = []
    for j in range(NCHUNK):
        copies.append(pltpu.async_copy(utab_hbm.at[u_idx.at[j]], u_rows.at[j], sem))
        copies.append(pltpu.async_copy(itab_hbm.at[p_idx.at[j]], p_rows.at[j], sem))
        copies.append(pltpu.async_copy(itab_hbm.at[n_idx.at[j]], n_rows.at[j], sem))
        copies.append(pltpu.async_copy(mu_hbm.at[us_idx.at[j]], mu_rows.at[j], sem))
        copies.append(pltpu.async_copy(mi_hbm.at[ps_idx.at[j]], mp_rows.at[j], sem))
        copies.append(pltpu.async_copy(mi_hbm.at[ns_idx.at[j]], mn_rows.at[j], sem))
    for c in copies:
        c.wait()

    iota = lax.iota(jnp.int32, L)

    def group(g, racc):
        c = lax.shift_right_logical(g, 3)
        rr = jnp.bitwise_and(g, 7) * L + iota
        cvec = jnp.zeros((L,), jnp.int32) + c

        accp = jnp.zeros((L,), jnp.float32)
        accn = jnp.zeros((L,), jnp.float32)
        for d in range(EMBED_DIM):
            dvec = jnp.full((L,), d, jnp.int32)
            u = plsc.load_gather(u_rows, [cvec, rr, dvec])
            p = plsc.load_gather(p_rows, [cvec, rr, dvec])
            n = plsc.load_gather(n_rows, [cvec, rr, dvec])
            dp = u - p
            dn = u - n
            accp = accp + dp * dp
            accn = accn + dn * dn

        iu = plsc.load_gather(u_idx, [cvec, rr])
        ip = plsc.load_gather(p_idx, [cvec, rr])
        in_ = plsc.load_gather(n_idx, [cvec, rr])
        mu = plsc.load_gather(mu_rows, [cvec, rr, jnp.bitwise_and(iu, 15)])
        mp = plsc.load_gather(mp_rows, [cvec, rr, jnp.bitwise_and(ip, 15)])
        mn = plsc.load_gather(mn_rows, [cvec, rr, jnp.bitwise_and(in_, 15)])

        lmu = _ln(jnp.maximum(mu, 0.0) + 1.0)
        lmp = _ln(jnp.maximum(mp, 0.0) + 1.0)
        lmn = _ln(jnp.maximum(mn, 0.0) + 1.0)
        dpos = LAM * _ln(accp + 0.01)
        dneg = LAM * _ln(accn + 0.01)

        sp = _sigmoid(lmu * lmp - dpos)
        sn = _sigmoid(lmu * lmn - dneg)

        pos_v[pl.ds(g * L, L)] = sp
        neg_v[pl.ds(g * L, L)] = sn

        return racc + mu * mu + mp * mp + mn * mn * (1.0 / BATCH)

    racc = lax.fori_loop(0, NGROUP, group, jnp.zeros((L,), jnp.float32))

    reg_v[...] = jnp.zeros((L,), jnp.float32) + jnp.sum(racc)

    pltpu.sync_copy(pos_v, pos_out.at[pl.ds(base, BPW)])
    pltpu.sync_copy(neg_v, neg_out.at[pl.ds(base, BPW)])
    pltpu.sync_copy(reg_v, reg_out.at[wid])


@jax.jit
def _run(users, pos, neg, user_table, item_table, mass_u, mass_i):
    mesh = plsc.VectorSubcoreMesh(core_axis_name="c", subcore_axis_name="s")
    k = pl.kernel(
        _sc_body,
        out_type=[
            jax.ShapeDtypeStruct((BATCH,), jnp.float32),
            jax.ShapeDtypeStruct((BATCH,), jnp.float32),
            jax.ShapeDtypeStruct((NW, L), jnp.float32),
        ],
        mesh=mesh,
        compiler_params=pltpu.CompilerParams(
            needs_layout_passes=False, use_tc_tiling_on_sc=False),
        scratch_types=[
            pltpu.VMEM((NCHUNK, CHUNK), jnp.int32),
            pltpu.VMEM((NCHUNK, CHUNK), jnp.int32),
            pltpu.VMEM((NCHUNK, CHUNK), jnp.int32),
            pltpu.VMEM((NCHUNK, CHUNK), jnp.int32),
            pltpu.VMEM((NCHUNK, CHUNK), jnp.int32),
            pltpu.VMEM((NCHUNK, CHUNK), jnp.int32),
            pltpu.VMEM((NCHUNK, CHUNK, EMBED_DIM), jnp.float32),
            pltpu.VMEM((NCHUNK, CHUNK, EMBED_DIM), jnp.float32),
            pltpu.VMEM((NCHUNK, CHUNK, EMBED_DIM), jnp.float32),
            pltpu.VMEM((NCHUNK, CHUNK, L), jnp.float32),
            pltpu.VMEM((NCHUNK, CHUNK, L), jnp.float32),
            pltpu.VMEM((NCHUNK, CHUNK, L), jnp.float32),
            pltpu.VMEM((BPW,), jnp.float32),
            pltpu.VMEM((BPW,), jnp.float32),
            pltpu.VMEM((L,), jnp.float32),
            pltpu.SemaphoreType.DMA,
        ],
    )
    one = 1.0 + 0.0 * users[0].astype(jnp.float32)
    ut = user_table * one
    it = item_table * one
    mu2 = mass_u.reshape(N_USERS // L, L)
    mi2 = mass_i.reshape(N_ITEMS // L, L)
    pos_s, neg_s, regp = k(users, pos, neg, ut, it, mu2, mi2)
    reg_loss = 0.5 * jnp.sum(regp[:, 0])
    return pos_s.reshape(BATCH, 1), neg_s.reshape(BATCH, 1), reg_loss


def kernel(users, pos, neg, user_table, item_table, mass_u, mass_i):
    return _run(users.astype(jnp.int32), pos.astype(jnp.int32),
                neg.astype(jnp.int32), user_table, item_table,
                mass_u, mass_i)


# speedup vs baseline: 1.8858x; 1.8858x over previous
import jax
import jax.numpy as jnp
from jax import lax
from jax.experimental import pallas as pl
from jax.experimental.pallas import tpu as pltpu
from jax.experimental.pallas import tpu_sc as plsc

N_USERS = 1000000
N_ITEMS = 1000000
EMBED_DIM = 16
BATCH = 16384
LAM = 1.0

NC = 2
NS = 16
L = 16
NW = NC * NS
BPW = BATCH // NW
NCHUNK = 4
CHUNK = BPW // NCHUNK
NGROUP = BPW // L

_LN2 = 0.69314718
_SQRT2 = 1.4142135


def _ln(x):
    xi = lax.bitcast_convert_type(x, jnp.int32)
    e = lax.shift_right_arithmetic(xi, 23) - 127
    mi = jnp.bitwise_or(jnp.bitwise_and(xi, 0x007FFFFF), 0x3F800000)
    m = lax.bitcast_convert_type(mi, jnp.float32)
    big = m > _SQRT2
    m = jnp.where(big, m * 0.5, m)
    e = jnp.where(big, e + 1, e)
    ef = e.astype(jnp.float32)
    s = (m - 1.0) / (m + 1.0)
    z = s * s
    p = 2.0 * s * (1.0 + z * (1.0 / 3.0 + z * (0.2 + z * (1.0 / 7.0 + z * (1.0 / 9.0)))))
    return ef * _LN2 + p


def _sigmoid(g):
    return 1.0 / (1.0 + jnp.exp(-g))


def _sc_body(users_hbm, pos_hbm, neg_hbm, utab_hbm, itab_hbm,
             mu_hbm, mi_hbm,
             pos_out, neg_out, reg_out,
             u_idx, p_idx, n_idx,
             us_idx, ps_idx, ns_idx,
             u_rows, p_rows, n_rows,
             mu_rows, mp_rows, mn_rows,
             pos_v, neg_v, reg_v, sem):
    wid = lax.axis_index("s") * NC + lax.axis_index("c")
    base = wid * BPW

    for j in range(NCHUNK):
        off = base + j * CHUNK
        pltpu.sync_copy(users_hbm.at[pl.ds(off, CHUNK)], u_idx.at[j])
        pltpu.sync_copy(pos_hbm.at[pl.ds(off, CHUNK)], p_idx.at[j])
        pltpu.sync_copy(neg_hbm.at[pl.ds(off, CHUNK)], n_idx.at[j])

    for j in range(NCHUNK):
        for r in range(CHUNK // L):
            sl = pl.ds(r * L, L)
            us_idx[j, sl] = lax.shift_right_logical(u_idx[j, sl], 4)
            ps_idx[j, sl] = lax.shift_right_logical(p_idx[j, sl], 4)
            ns_idx[j, sl] = lax.shift_right_logical(n_idx[j, sl], 4)

    copies = []
    for j in range(NCHUNK):
        copies.append(pltpu.async_copy(utab_hbm.at[u_idx.at[j]], u_rows.at[j], sem))
        copies.append(pltpu.async_copy(itab_hbm.at[p_idx.at[j]], p_rows.at[j], sem))
        copies.append(pltpu.async_copy(itab_hbm.at[n_idx.at[j]], n_rows.at[j], sem))
        copies.append(pltpu.async_copy(mu_hbm.at[us_idx.at[j]], mu_rows.at[j], sem))
        copies.append(pltpu.async_copy(mi_hbm.at[ps_idx.at[j]], mp_rows.at[j], sem))
        copies.append(pltpu.async_copy(mi_hbm.at[ns_idx.at[j]], mn_rows.at[j], sem))
    for c in copies:
        c.wait()

    iota = lax.iota(jnp.int32, L)

    def group(g, racc):
        c = lax.shift_right_logical(g, 3)
        rr = jnp.bitwise_and(g, 7) * L + iota
        cvec = jnp.zeros((L,), jnp.int32) + c

        accp = jnp.zeros((L,), jnp.float32)
        accn = jnp.zeros((L,), jnp.float32)
        for d in range(EMBED_DIM):
            dvec = jnp.full((L,), d, jnp.int32)
            u = plsc.load_gather(u_rows, [cvec, rr, dvec])
            p = plsc.load_gather(p_rows, [cvec, rr, dvec])
            n = plsc.load_gather(n_rows, [cvec, rr, dvec])
            dp = u - p
            dn = u - n
            accp = accp + dp * dp
            accn = accn + dn * dn

        iu = plsc.load_gather(u_idx, [cvec, rr])
        ip = plsc.load_gather(p_idx, [cvec, rr])
        in_ = plsc.load_gather(n_idx, [cvec, rr])
        mu = plsc.load_gather(mu_rows, [cvec, rr, jnp.bitwise_and(iu, 15)])
        mp = plsc.load_gather(mp_rows, [cvec, rr, jnp.bitwise_and(ip, 15)])
        mn = plsc.load_gather(mn_rows, [cvec, rr, jnp.bitwise_and(in_, 15)])

        lmu = _ln(jnp.maximum(mu, 0.0) + 1.0)
        lmp = _ln(jnp.maximum(mp, 0.0) + 1.0)
        lmn = _ln(jnp.maximum(mn, 0.0) + 1.0)
        dpos = LAM * _ln(accp + 0.01)
        dneg = LAM * _ln(accn + 0.01)

        sp = _sigmoid(lmu * lmp - dpos)
        sn = _sigmoid(lmu * lmn - dneg)

        pos_v[pl.ds(g * L, L)] = sp
        neg_v[pl.ds(g * L, L)] = sn

        return racc + mu * mu + mp * mp + mn * mn * (1.0 / BATCH)

    racc = lax.fori_loop(0, NGROUP, group, jnp.zeros((L,), jnp.float32))

    reg_v[...] = jnp.zeros((L,), jnp.float32) + jnp.sum(racc)

    pltpu.sync_copy(pos_v, pos_out.at[pl.ds(base, BPW)])
    pltpu.sync_copy(neg_v, neg_out.at[pl.ds(base, BPW)])
    pltpu.sync_copy(reg_v, reg_out.at[wid])


@jax.jit
def _run(users, pos, neg, user_table, item_table, mass_u, mass_i):
    mesh = plsc.VectorSubcoreMesh(core_axis_name="c", subcore_axis_name="s")
    k = pl.kernel(
        _sc_body,
        out_type=[
            jax.ShapeDtypeStruct((BATCH,), jnp.float32),
            jax.ShapeDtypeStruct((BATCH,), jnp.float32),
            jax.ShapeDtypeStruct((NW, L), jnp.float32),
        ],
        mesh=mesh,
        compiler_params=pltpu.CompilerParams(
            needs_layout_passes=False, use_tc_tiling_on_sc=False),
        scratch_types=[
            pltpu.VMEM((NCHUNK, CHUNK), jnp.int32),
            pltpu.VMEM((NCHUNK, CHUNK), jnp.int32),
            pltpu.VMEM((NCHUNK, CHUNK), jnp.int32),
            pltpu.VMEM((NCHUNK, CHUNK), jnp.int32),
            pltpu.VMEM((NCHUNK, CHUNK), jnp.int32),
            pltpu.VMEM((NCHUNK, CHUNK), jnp.int32),
            pltpu.VMEM((NCHUNK, CHUNK, EMBED_DIM), jnp.float32),
            pltpu.VMEM((NCHUNK, CHUNK, EMBED_DIM), jnp.float32),
            pltpu.VMEM((NCHUNK, CHUNK, EMBED_DIM), jnp.float32),
            pltpu.VMEM((NCHUNK, CHUNK, L), jnp.float32),
            pltpu.VMEM((NCHUNK, CHUNK, L), jnp.float32),
            pltpu.VMEM((NCHUNK, CHUNK, L), jnp.float32),
            pltpu.VMEM((BPW,), jnp.float32),
            pltpu.VMEM((BPW,), jnp.float32),
            pltpu.VMEM((L,), jnp.float32),
            pltpu.SemaphoreType.DMA,
        ],
    )
    mu2 = mass_u.reshape(N_USERS // L, L)
    mi2 = mass_i.reshape(N_ITEMS // L, L)
    pos_s, neg_s, regp = k(users, pos, neg, user_table, item_table,
                           mu2, mi2)
    reg_loss = 0.5 * jnp.sum(regp[:, 0])
    return pos_s.reshape(BATCH, 1), neg_s.reshape(BATCH, 1), reg_loss


def kernel(users, pos, neg, user_table, item_table, mass_u, mass_i):
    return _run(users.astype(jnp.int32), pos.astype(jnp.int32),
                neg.astype(jnp.int32), user_table, item_table,
                mass_u, mass_i)
